# baseline (speedup 1.0000x reference)
import functools
import jax
import jax.numpy as jnp
from jax import lax
from jax.experimental import pallas as pl
from jax.experimental.pallas import tpu as pltpu
from jax.experimental.pallas import tpu_sc as plsc

_BLK = 4096
_L = 16


def _sc_select(p0, p1, p2, idx):
    info = plsc.get_sparse_core_info()
    nw = info.num_cores * info.num_subcores
    b = idx.shape[0]
    bpw = b // nw

    mesh = plsc.VectorSubcoreMesh(core_axis_name="c", subcore_axis_name="s")

    @functools.partial(
        pl.kernel,
        mesh=mesh,
        out_type=jax.ShapeDtypeStruct((b,), jnp.float32),
        scratch_types=[
            pltpu.VMEM((bpw,), jnp.float32),
            pltpu.VMEM((bpw,), jnp.float32),
            pltpu.VMEM((bpw,), jnp.float32),
            pltpu.VMEM((bpw,), jnp.int32),
            pltpu.VMEM((bpw,), jnp.float32),
        ],
        compiler_params=pltpu.CompilerParams(needs_layout_passes=False),
    )
    def k(p0_hbm, p1_hbm, p2_hbm, idx_hbm, out_hbm,
          p0_v, p1_v, p2_v, idx_v, out_v):
        wid = lax.axis_index("s") * info.num_cores + lax.axis_index("c")
        base = wid * bpw
        pltpu.sync_copy(p0_hbm.at[pl.ds(base, bpw)], p0_v)
        pltpu.sync_copy(p1_hbm.at[pl.ds(base, bpw)], p1_v)
        pltpu.sync_copy(p2_hbm.at[pl.ds(base, bpw)], p2_v)
        pltpu.sync_copy(idx_hbm.at[pl.ds(base, bpw)], idx_v)

        @pl.loop(0, bpw // _L)
        def _(j):
            s = pl.ds(j * _L, _L)
            iv = idx_v[s]
            out_v[s] = jnp.where(iv == 0, p0_v[s],
                                 jnp.where(iv == 1, p1_v[s], p2_v[s]))

        pltpu.sync_copy(out_v, out_hbm.at[pl.ds(base, bpw)])

    return k(p0, p1, p2, idx)


def _tc_body(x_ref, w1_ref, w2_ref, wall_ref, out0_ref, out1_ref, out2_ref):
    b1 = wall_ref[153:154, :]
    b2 = wall_ref[154:155, :32]
    f = jnp.maximum(x_ref[...] @ w1_ref[...] + b1, 0.0)
    f = jnp.maximum(f @ w2_ref[...] + b2, 0.0)
    outs = (out0_ref, out1_ref, out2_ref)
    for i in range(3):
        w3a = wall_ref[i * 48:i * 48 + 32, :]
        w3b = wall_ref[i * 48 + 32:i * 48 + 48, :]
        embr = wall_ref[144 + i:145 + i, :16]
        b3r = wall_ref[147 + i:148 + i, :]
        w4r = wall_ref[150 + i:151 + i, :]
        b4s = wall_ref[155:156, i:i + 1]
        t = embr @ w3b + b3r
        h = jnp.maximum(f @ w3a + t, 0.0)
        row = lax.dot_general(w4r, h, (((1,), (1,)), ((), ()))) + b4s
        outs[i][...] = row.reshape(-1)


def _tc_call(x, w1, w2, wall):
    bsz = x.shape[0]
    full = lambda i: (0, 0)
    return pl.pallas_call(
        _tc_body,
        grid=(bsz // _BLK,),
        in_specs=[
            pl.BlockSpec((_BLK, 128), lambda i: (i, 0)),
            pl.BlockSpec((128, 64), full),
            pl.BlockSpec((64, 32), full),
            pl.BlockSpec((160, 64), full),
        ],
        out_specs=[pl.BlockSpec((_BLK,), lambda i: (i,))] * 3,
        out_shape=[jax.ShapeDtypeStruct((bsz,), jnp.float32)] * 3,
        compiler_params=pltpu.CompilerParams(
            dimension_semantics=("arbitrary",)),
    )(x, w1, w2, wall)


def kernel(x, regime_ids, W1, b1, W2, b2, emb, W3, b3, W4, b4):
    idx = regime_ids.astype(jnp.int32)
    wall = jnp.zeros((160, 64), jnp.float32)
    wall = wall.at[:144].set(W3.reshape(144, 64))
    wall = wall.at[144:147, :16].set(emb)
    wall = wall.at[147:150].set(b3)
    wall = wall.at[150:153].set(W4[:, :, 0])
    wall = wall.at[153].set(b1)
    wall = wall.at[154, :32].set(b2)
    wall = wall.at[155, :3].set(b4[:, 0])
    p0, p1, p2 = _tc_call(x, W1, W2, wall)
    return _sc_select(p0, p1, p2, idx).reshape(-1, 1)
